# Initial kernel scaffold; baseline (speedup 1.0000x reference)
import jax
import jax.numpy as jnp
from jax import lax
from jax.experimental import pallas as pl
from jax.experimental.pallas import tpu as pltpu
from jax.experimental.pallas import tpu_sc as plsc

_N_NODES = 10000
_E_TOTAL = 320000
_N_PAD = 10240
_NC, _NS = 2, 16
_NW = _NC * _NS
_EPW = _E_TOTAL // _NW
_CHUNK = 80
_NCHUNK = _EPW // _CHUNK
_ROWS_PT = _N_PAD // _NS


def _sc_propagate(feat_dim, with_degrees):
  mesh = plsc.VectorSubcoreMesh(
      core_axis_name="c", subcore_axis_name="s",
      num_cores=_NC, num_subcores=_NS)
  out_type = [jax.ShapeDtypeStruct((_NC * _N_PAD, feat_dim), jnp.float32)]
  if with_degrees:
    out_type += [jax.ShapeDtypeStruct((_NC * _N_PAD,), jnp.float32)] * 2
  scratch = [
      pltpu.VMEM((_CHUNK,), jnp.int32),
      pltpu.VMEM((_CHUNK,), jnp.int32),
      pltpu.VMEM((_CHUNK, feat_dim), jnp.float32),
      pltpu.VMEM_SHARED((_N_PAD, feat_dim), jnp.float32),
      pltpu.SemaphoreType.DMA,
  ]
  if with_degrees:
    scratch += [
        pltpu.VMEM((_CHUNK,), jnp.float32),
        pltpu.VMEM((_ROWS_PT,), jnp.float32),
        pltpu.VMEM_SHARED((_N_PAD,), jnp.float32),
        pltpu.VMEM_SHARED((_N_PAD,), jnp.float32),
    ]

  def body(table, gsrc, ssrc, *rest):
    if with_degrees:
      (acc_out, dg_out, ds_out,
       gidx, sidx, rows, acc, sem, ones, zvec, dg, ds) = rest
    else:
      acc_out, gidx, sidx, rows, acc, sem = rest
    cid = lax.axis_index("c")
    sid = lax.axis_index("s")
    wid = sid * _NC + cid
    zero16 = jnp.zeros((16,), jnp.float32)

    def zrow(i, carry):
      for j in range(feat_dim // 16):
        rows[i, pl.ds(j * 16, 16)] = zero16
      return carry
    lax.fori_loop(0, _CHUNK, zrow, 0)
    row0 = sid * _ROWS_PT
    for k in range(_ROWS_PT // _CHUNK):
      pltpu.sync_copy(rows, acc.at[pl.ds(row0 + k * _CHUNK, _CHUNK)])
    if with_degrees:
      def zv(i, carry):
        zvec[pl.ds(i * 16, 16)] = zero16
        return carry
      lax.fori_loop(0, _ROWS_PT // 16, zv, 0)
      for j in range(_CHUNK // 16):
        ones[pl.ds(j * 16, 16)] = jnp.ones((16,), jnp.float32)
      pltpu.sync_copy(zvec, dg.at[pl.ds(row0, _ROWS_PT)])
      pltpu.sync_copy(zvec, ds.at[pl.ds(row0, _ROWS_PT)])
    plsc.subcore_barrier()

    ebase = wid * _EPW
    def step(k, carry):
      b = ebase + k * _CHUNK
      pltpu.sync_copy(gsrc.at[pl.ds(b, _CHUNK)], gidx)
      pltpu.sync_copy(ssrc.at[pl.ds(b, _CHUNK)], sidx)
      pltpu.async_copy(table.at[gidx], rows, sem).wait()
      pltpu.sync_copy(rows, acc.at[sidx], add=True)
      if with_degrees:
        pltpu.sync_copy(ones, dg.at[gidx], add=True)
        pltpu.sync_copy(ones, ds.at[sidx], add=True)
      return carry
    lax.fori_loop(0, _NCHUNK, step, 0)
    plsc.subcore_barrier()

    obase = cid * _N_PAD + row0
    pltpu.sync_copy(acc.at[pl.ds(row0, _ROWS_PT)],
                    acc_out.at[pl.ds(obase, _ROWS_PT)])
    if with_degrees:
      pltpu.sync_copy(dg.at[pl.ds(row0, _ROWS_PT)],
                      dg_out.at[pl.ds(obase, _ROWS_PT)])
      pltpu.sync_copy(ds.at[pl.ds(row0, _ROWS_PT)],
                      ds_out.at[pl.ds(obase, _ROWS_PT)])

  return pl.kernel(
      body,
      out_type=tuple(out_type) if with_degrees else out_type[0],
      mesh=mesh,
      scratch_types=tuple(scratch))


_sc_prop_deg = _sc_propagate(128, True)
_sc_prop128 = _sc_propagate(128, False)
_sc_prop32 = _sc_propagate(32, False)


def _tc_matmul(x, w):
  m, k = x.shape
  n = w.shape[1]
  br = 1000 if m % 1000 == 0 else 1024

  def body(xr, wr, outr):
    outr[...] = jnp.dot(xr[...], wr[...], preferred_element_type=jnp.float32)

  return pl.pallas_call(
      body,
      grid=(m // br,),
      in_specs=[pl.BlockSpec((br, k), lambda i: (i, 0)),
                pl.BlockSpec((k, n), lambda i: (0, 0))],
      out_specs=pl.BlockSpec((br, n), lambda i: (i, 0)),
      out_shape=jax.ShapeDtypeStruct((m, n), jnp.float32),
  )(x, w)


def _tc_edge_scale(acc, deg, feat_dim):
  br = 1024
  acc3 = acc.reshape(_NC, _N_PAD, feat_dim)
  deg3 = deg.reshape(_NC, _N_PAD, 1)

  def body(a0, a1, d0, d1, o):
    s = a0[0] + a1[0]
    b = d0[0] + d1[0]
    o[...] = jnp.where(b > 0, 1.0 / b, 0.0) * s

  return pl.pallas_call(
      body,
      grid=(_N_PAD // br,),
      in_specs=[pl.BlockSpec((1, br, feat_dim), lambda i: (0, i, 0)),
                pl.BlockSpec((1, br, feat_dim), lambda i: (1, i, 0)),
                pl.BlockSpec((1, br, 1), lambda i: (0, i, 0)),
                pl.BlockSpec((1, br, 1), lambda i: (1, i, 0))],
      out_specs=pl.BlockSpec((br, feat_dim), lambda i: (i, 0)),
      out_shape=jax.ShapeDtypeStruct((_N_PAD, feat_dim), jnp.float32),
  )(acc3, acc3, deg3, deg3)


def _tc_node_out(acc, deg, bias, w, feat_dim):
  br = 1024
  acc3 = acc.reshape(_NC, _N_PAD, feat_dim)
  deg3 = deg.reshape(_NC, _N_PAD, 1)
  bias2 = bias.reshape(1, feat_dim)
  n_out = feat_dim if w is None else w.shape[1]

  def body(a0, a1, d0, d1, br_, *rest):
    if w is None:
      (o,) = rest
    else:
      wr, o = rest
    s = a0[0] + a1[0]
    d = d0[0] + d1[0]
    h = jnp.maximum(jnp.where(d > 0, 1.0 / d, 0.0) * s + br_[...], 0.0)
    if w is None:
      o[...] = h
    else:
      o[...] = jnp.dot(h, wr[...], preferred_element_type=jnp.float32)

  in_specs = [pl.BlockSpec((1, br, feat_dim), lambda i: (0, i, 0)),
              pl.BlockSpec((1, br, feat_dim), lambda i: (1, i, 0)),
              pl.BlockSpec((1, br, 1), lambda i: (0, i, 0)),
              pl.BlockSpec((1, br, 1), lambda i: (1, i, 0)),
              pl.BlockSpec((1, feat_dim), lambda i: (0, 0))]
  args = [acc3, acc3, deg3, deg3, bias2]
  if w is not None:
    in_specs.append(pl.BlockSpec(w.shape, lambda i: (0, 0)))
    args.append(w)
  return pl.pallas_call(
      body,
      grid=(_N_PAD // br,),
      in_specs=in_specs,
      out_specs=pl.BlockSpec((br, n_out), lambda i: (i, 0)),
      out_shape=jax.ShapeDtypeStruct((_N_PAD, n_out), jnp.float32),
  )(*args)


def kernel(x, edge, W1, b1, W2, b2):
  nidx = edge[0]
  hidx = edge[1]

  xl1 = _tc_matmul(x, W1)
  acc1, d_node, d_hedge = _sc_prop_deg(xl1, nidx, hidx)
  ef1 = _tc_edge_scale(acc1, d_hedge, 128)
  acc2 = _sc_prop128(ef1, hidx, nidx)
  xl2 = _tc_node_out(acc2, d_node, b1, W2, 128)

  acc3 = _sc_prop32(xl2, nidx, hidx)
  ef2 = _tc_edge_scale(acc3, d_hedge, 32)
  acc4 = _sc_prop32(ef2, hidx, nidx)
  out = _tc_node_out(acc4, d_node, b2, None, 32)
  return out[:_N_NODES]


# speedup vs baseline: 9.0329x; 9.0329x over previous
import jax
import jax.numpy as jnp
from jax import lax
from jax.experimental import pallas as pl
from jax.experimental.pallas import tpu as pltpu
from jax.experimental.pallas import tpu_sc as plsc

_N_NODES = 10000
_E_TOTAL = 320000
_N_PAD = 10240
_NC, _NS = 2, 16
_NW = _NC * _NS
_EPW = _E_TOTAL // _NW
_CHUNK = 80
_NCHUNK = _EPW // _CHUNK
_ROWS_PT = _N_PAD // _NS


def _sc_propagate(feat_dim, with_degrees):
  mesh = plsc.VectorSubcoreMesh(
      core_axis_name="c", subcore_axis_name="s",
      num_cores=_NC, num_subcores=_NS)
  out_type = [jax.ShapeDtypeStruct((_NC * _N_PAD, feat_dim), jnp.float32)]
  if with_degrees:
    out_type += [jax.ShapeDtypeStruct((_NC * _N_PAD,), jnp.float32)] * 2
  scratch = [
      pltpu.VMEM((_CHUNK,), jnp.int32),
      pltpu.VMEM((_CHUNK,), jnp.int32),
      pltpu.VMEM((_CHUNK, feat_dim), jnp.float32),
      pltpu.VMEM_SHARED((_N_PAD, feat_dim), jnp.float32),
      pltpu.SemaphoreType.DMA,
  ]
  if with_degrees:
    scratch += [
        pltpu.VMEM((_CHUNK,), jnp.float32),
        pltpu.VMEM((_ROWS_PT,), jnp.float32),
        pltpu.VMEM_SHARED((_N_PAD,), jnp.float32),
        pltpu.VMEM_SHARED((_N_PAD,), jnp.float32),
    ]

  def body(table, gsrc, ssrc, *rest):
    if with_degrees:
      (acc_out, dg_out, ds_out,
       gidx, sidx, rows, acc, sem, ones, zvec, dg, ds) = rest
    else:
      acc_out, gidx, sidx, rows, acc, sem = rest
    cid = lax.axis_index("c")
    sid = lax.axis_index("s")
    wid = sid * _NC + cid
    zero16 = jnp.zeros((16,), jnp.float32)

    def zrow(i, carry):
      for j in range(feat_dim // 16):
        rows[i, pl.ds(j * 16, 16)] = zero16
      return carry
    lax.fori_loop(0, _CHUNK, zrow, 0)
    row0 = sid * _ROWS_PT
    for k in range(_ROWS_PT // _CHUNK):
      pltpu.sync_copy(rows, acc.at[pl.ds(row0 + k * _CHUNK, _CHUNK)])
    if with_degrees:
      def zv(i, carry):
        zvec[pl.ds(i * 16, 16)] = zero16
        return carry
      lax.fori_loop(0, _ROWS_PT // 16, zv, 0)
      for j in range(_CHUNK // 16):
        ones[pl.ds(j * 16, 16)] = jnp.ones((16,), jnp.float32)
      pltpu.sync_copy(zvec, dg.at[pl.ds(row0, _ROWS_PT)])
      pltpu.sync_copy(zvec, ds.at[pl.ds(row0, _ROWS_PT)])
    plsc.subcore_barrier()

    ebase = wid * _EPW
    def step(k, carry):
      b = ebase + k * _CHUNK
      pltpu.sync_copy(gsrc.at[pl.ds(b, _CHUNK)], gidx)
      pltpu.sync_copy(ssrc.at[pl.ds(b, _CHUNK)], sidx)
      pltpu.async_copy(table.at[gidx], rows, sem).wait()
      pltpu.sync_copy(rows, acc.at[sidx], add=True)
      if with_degrees:
        pltpu.sync_copy(ones, dg.at[gidx], add=True)
        pltpu.sync_copy(ones, ds.at[sidx], add=True)
      return carry
    lax.fori_loop(0, _NCHUNK, step, 0)
    plsc.subcore_barrier()

    obase = cid * _N_PAD + row0
    pltpu.sync_copy(acc.at[pl.ds(row0, _ROWS_PT)],
                    acc_out.at[pl.ds(obase, _ROWS_PT)])
    if with_degrees:
      pltpu.sync_copy(dg.at[pl.ds(row0, _ROWS_PT)],
                      dg_out.at[pl.ds(obase, _ROWS_PT)])
      pltpu.sync_copy(ds.at[pl.ds(row0, _ROWS_PT)],
                      ds_out.at[pl.ds(obase, _ROWS_PT)])

  return pl.kernel(
      body,
      out_type=tuple(out_type) if with_degrees else out_type[0],
      mesh=mesh,
      scratch_types=tuple(scratch),
      compiler_params=pltpu.CompilerParams(
          use_tc_tiling_on_sc=False if feat_dim < 128 else None))


_sc_prop_deg = _sc_propagate(128, True)
_sc_prop128 = _sc_propagate(128, False)
_sc_prop32 = _sc_propagate(32, False)


def _tc_matmul(x, w):
  m, k = x.shape
  n = w.shape[1]
  br = 1000 if m % 1000 == 0 else 1024

  def body(xr, wr, outr):
    outr[...] = jnp.dot(xr[...], wr[...], preferred_element_type=jnp.float32)

  return pl.pallas_call(
      body,
      grid=(m // br,),
      in_specs=[pl.BlockSpec((br, k), lambda i: (i, 0)),
                pl.BlockSpec((k, n), lambda i: (0, 0))],
      out_specs=pl.BlockSpec((br, n), lambda i: (i, 0)),
      out_shape=jax.ShapeDtypeStruct((m, n), jnp.float32),
  )(x, w)


def _tc_edge_scale(acc, deg, feat_dim):
  br = 1024
  acc3 = acc.reshape(_NC, _N_PAD, feat_dim)
  deg3 = deg.reshape(_NC, _N_PAD, 1)

  def body(a0, a1, d0, d1, o):
    s = a0[0] + a1[0]
    b = d0[0] + d1[0]
    o[...] = jnp.where(b > 0, 1.0 / b, 0.0) * s

  return pl.pallas_call(
      body,
      grid=(_N_PAD // br,),
      in_specs=[pl.BlockSpec((1, br, feat_dim), lambda i: (0, i, 0)),
                pl.BlockSpec((1, br, feat_dim), lambda i: (1, i, 0)),
                pl.BlockSpec((1, br, 1), lambda i: (0, i, 0)),
                pl.BlockSpec((1, br, 1), lambda i: (1, i, 0))],
      out_specs=pl.BlockSpec((br, feat_dim), lambda i: (i, 0)),
      out_shape=jax.ShapeDtypeStruct((_N_PAD, feat_dim), jnp.float32),
  )(acc3, acc3, deg3, deg3)


def _tc_node_out(acc, deg, bias, w, feat_dim):
  br = 1024
  acc3 = acc.reshape(_NC, _N_PAD, feat_dim)
  deg3 = deg.reshape(_NC, _N_PAD, 1)
  bias2 = bias.reshape(1, feat_dim)
  n_out = feat_dim if w is None else w.shape[1]

  def body(a0, a1, d0, d1, br_, *rest):
    if w is None:
      (o,) = rest
    else:
      wr, o = rest
    s = a0[0] + a1[0]
    d = d0[0] + d1[0]
    h = jnp.maximum(jnp.where(d > 0, 1.0 / d, 0.0) * s + br_[...], 0.0)
    if w is None:
      o[...] = h
    else:
      o[...] = jnp.dot(h, wr[...], preferred_element_type=jnp.float32)

  in_specs = [pl.BlockSpec((1, br, feat_dim), lambda i: (0, i, 0)),
              pl.BlockSpec((1, br, feat_dim), lambda i: (1, i, 0)),
              pl.BlockSpec((1, br, 1), lambda i: (0, i, 0)),
              pl.BlockSpec((1, br, 1), lambda i: (1, i, 0)),
              pl.BlockSpec((1, feat_dim), lambda i: (0, 0))]
  args = [acc3, acc3, deg3, deg3, bias2]
  if w is not None:
    in_specs.append(pl.BlockSpec(w.shape, lambda i: (0, 0)))
    args.append(w)
  return pl.pallas_call(
      body,
      grid=(_N_PAD // br,),
      in_specs=in_specs,
      out_specs=pl.BlockSpec((br, n_out), lambda i: (i, 0)),
      out_shape=jax.ShapeDtypeStruct((_N_PAD, n_out), jnp.float32),
  )(*args)


def kernel(x, edge, W1, b1, W2, b2):
  nidx = edge[0]
  hidx = edge[1]

  xl1 = _tc_matmul(x, W1)
  acc1, d_node, d_hedge = _sc_prop_deg(xl1, nidx, hidx)
  ef1 = _tc_edge_scale(acc1, d_hedge, 128)
  acc2 = _sc_prop128(ef1, hidx, nidx)
  xl2 = _tc_node_out(acc2, d_node, b1, W2, 128)

  acc3 = _sc_prop32(xl2, nidx, hidx)
  ef2 = _tc_edge_scale(acc3, d_hedge, 32)
  acc4 = _sc_prop32(ef2, hidx, nidx)
  out = _tc_node_out(acc4, d_node, b2, None, 32)
  return out[:_N_NODES]


# speedup vs baseline: 21.6732x; 2.3994x over previous
import jax
import jax.numpy as jnp
from jax import lax
from jax.experimental import pallas as pl
from jax.experimental.pallas import tpu as pltpu
from jax.experimental.pallas import tpu_sc as plsc

_N_NODES = 10000
_E_TOTAL = 320000
_N_PAD = 10240
_NC, _NS = 2, 16
_NW = _NC * _NS
_EPW = _E_TOTAL // _NW
_CHUNK = 80
_CPW = _EPW // _CHUNK
_NBUF = 2
_NGRP = -(-_CPW // _NBUF)
_ROWS_PT = _N_PAD // _NS


def _sc_propagate(feat_dim, with_degrees):
  mesh = plsc.VectorSubcoreMesh(
      core_axis_name="c", subcore_axis_name="s",
      num_cores=_NC, num_subcores=_NS)
  out_type = [jax.ShapeDtypeStruct((_NC * _N_PAD, feat_dim), jnp.float32)]
  if with_degrees:
    out_type += [jax.ShapeDtypeStruct((_NC * _N_PAD,), jnp.float32)] * 2
  scratch = [
      pltpu.VMEM((_CPW, _CHUNK), jnp.int32),
      pltpu.VMEM((_CPW, _CHUNK), jnp.int32),
      pltpu.VMEM((_NBUF, _CHUNK, feat_dim), jnp.float32),
      pltpu.VMEM_SHARED((_N_PAD, feat_dim), jnp.float32),
  ] + [pltpu.SemaphoreType.DMA] * _NBUF
  if with_degrees:
    scratch += [
        pltpu.VMEM((_CHUNK,), jnp.float32),
        pltpu.VMEM((_ROWS_PT,), jnp.float32),
        pltpu.VMEM_SHARED((_N_PAD,), jnp.float32),
        pltpu.VMEM_SHARED((_N_PAD,), jnp.float32),
    ]

  def body(table, gsrc, ssrc, *rest):
    if with_degrees:
      (acc_out, dg_out, ds_out, gidx2, sidx2, rows3, acc,
       *sems, ones, zvec, dg, ds) = rest
    else:
      acc_out, gidx2, sidx2, rows3, acc, *sems = rest
    cid = lax.axis_index("c")
    sid = lax.axis_index("s")
    wid = sid * _NC + cid
    zero16 = jnp.zeros((16,), jnp.float32)

    pltpu.sync_copy(gsrc.at[wid], gidx2)
    pltpu.sync_copy(ssrc.at[wid], sidx2)

    def zrow(i, carry):
      for j in range(feat_dim // 16):
        rows3[0, i, pl.ds(j * 16, 16)] = zero16
      return carry
    lax.fori_loop(0, _CHUNK, zrow, 0)
    row0 = sid * _ROWS_PT
    for k in range(_ROWS_PT // _CHUNK):
      pltpu.sync_copy(rows3.at[0], acc.at[pl.ds(row0 + k * _CHUNK, _CHUNK)])
    rem = _ROWS_PT - (_ROWS_PT // _CHUNK) * _CHUNK
    if rem:
      pltpu.sync_copy(rows3.at[0, pl.ds(0, rem)],
                      acc.at[pl.ds(row0 + _ROWS_PT - rem, rem)])
    if with_degrees:
      def zv(i, carry):
        zvec[pl.ds(i * 16, 16)] = zero16
        return carry
      lax.fori_loop(0, _ROWS_PT // 16, zv, 0)
      for j in range(_CHUNK // 16 + 1):
        o = min(j * 16, _CHUNK - 16)
        ones[pl.ds(o, 16)] = jnp.ones((16,), jnp.float32)
      pltpu.sync_copy(zvec, dg.at[pl.ds(row0, _ROWS_PT)])
      pltpu.sync_copy(zvec, ds.at[pl.ds(row0, _ROWS_PT)])
    plsc.subcore_barrier()

    for b in range(_NBUF):
      pltpu.async_copy(table.at[gidx2.at[b]], rows3.at[b], sems[b])

    def grp(g, carry):
      for b in range(_NBUF):
        k = g * _NBUF + b
        @pl.when(k < _CPW)
        def _():
          pltpu.make_async_copy(
              table.at[gidx2.at[b]], rows3.at[b], sems[b]).wait()
          pltpu.sync_copy(rows3.at[b], acc.at[sidx2.at[k]], add=True)
          if with_degrees:
            pltpu.sync_copy(ones, dg.at[gidx2.at[k]], add=True)
            pltpu.sync_copy(ones, ds.at[sidx2.at[k]], add=True)
          @pl.when(k + _NBUF < _CPW)
          def _():
            pltpu.async_copy(
                table.at[gidx2.at[k + _NBUF]], rows3.at[b], sems[b])
      return carry
    lax.fori_loop(0, _NGRP, grp, 0)
    plsc.subcore_barrier()

    obase = cid * _N_PAD + row0
    pltpu.sync_copy(acc.at[pl.ds(row0, _ROWS_PT)],
                    acc_out.at[pl.ds(obase, _ROWS_PT)])
    if with_degrees:
      pltpu.sync_copy(dg.at[pl.ds(row0, _ROWS_PT)],
                      dg_out.at[pl.ds(obase, _ROWS_PT)])
      pltpu.sync_copy(ds.at[pl.ds(row0, _ROWS_PT)],
                      ds_out.at[pl.ds(obase, _ROWS_PT)])

  return pl.kernel(
      body,
      out_type=tuple(out_type) if with_degrees else out_type[0],
      mesh=mesh,
      scratch_types=tuple(scratch),
      compiler_params=pltpu.CompilerParams(use_tc_tiling_on_sc=False))


_sc_prop_deg = _sc_propagate(128, True)
_sc_prop128 = _sc_propagate(128, False)
_sc_prop32 = _sc_propagate(32, False)


def _tc_matmul(x, w):
  m, k = x.shape
  n = w.shape[1]
  br = 1000 if m % 1000 == 0 else 1024

  def body(xr, wr, outr):
    outr[...] = jnp.dot(xr[...], wr[...], preferred_element_type=jnp.float32)

  return pl.pallas_call(
      body,
      grid=(m // br,),
      in_specs=[pl.BlockSpec((br, k), lambda i: (i, 0)),
                pl.BlockSpec((k, n), lambda i: (0, 0))],
      out_specs=pl.BlockSpec((br, n), lambda i: (i, 0)),
      out_shape=jax.ShapeDtypeStruct((m, n), jnp.float32),
  )(x, w)


def _tc_edge_scale(acc, deg, feat_dim):
  br = 1024
  acc3 = acc.reshape(_NC, _N_PAD, feat_dim)
  deg3 = deg.reshape(_NC, _N_PAD, 1)

  def body(a0, a1, d0, d1, o):
    s = a0[0] + a1[0]
    b = d0[0] + d1[0]
    o[...] = jnp.where(b > 0, 1.0 / b, 0.0) * s

  return pl.pallas_call(
      body,
      grid=(_N_PAD // br,),
      in_specs=[pl.BlockSpec((1, br, feat_dim), lambda i: (0, i, 0)),
                pl.BlockSpec((1, br, feat_dim), lambda i: (1, i, 0)),
                pl.BlockSpec((1, br, 1), lambda i: (0, i, 0)),
                pl.BlockSpec((1, br, 1), lambda i: (1, i, 0))],
      out_specs=pl.BlockSpec((br, feat_dim), lambda i: (i, 0)),
      out_shape=jax.ShapeDtypeStruct((_N_PAD, feat_dim), jnp.float32),
  )(acc3, acc3, deg3, deg3)


def _tc_node_out(acc, deg, bias, w, feat_dim):
  br = 1024
  acc3 = acc.reshape(_NC, _N_PAD, feat_dim)
  deg3 = deg.reshape(_NC, _N_PAD, 1)
  bias2 = bias.reshape(1, feat_dim)
  n_out = feat_dim if w is None else w.shape[1]

  def body(a0, a1, d0, d1, br_, *rest):
    if w is None:
      (o,) = rest
    else:
      wr, o = rest
    s = a0[0] + a1[0]
    d = d0[0] + d1[0]
    h = jnp.maximum(jnp.where(d > 0, 1.0 / d, 0.0) * s + br_[...], 0.0)
    if w is None:
      o[...] = h
    else:
      o[...] = jnp.dot(h, wr[...], preferred_element_type=jnp.float32)

  in_specs = [pl.BlockSpec((1, br, feat_dim), lambda i: (0, i, 0)),
              pl.BlockSpec((1, br, feat_dim), lambda i: (1, i, 0)),
              pl.BlockSpec((1, br, 1), lambda i: (0, i, 0)),
              pl.BlockSpec((1, br, 1), lambda i: (1, i, 0)),
              pl.BlockSpec((1, feat_dim), lambda i: (0, 0))]
  args = [acc3, acc3, deg3, deg3, bias2]
  if w is not None:
    in_specs.append(pl.BlockSpec(w.shape, lambda i: (0, 0)))
    args.append(w)
  return pl.pallas_call(
      body,
      grid=(_N_PAD // br,),
      in_specs=in_specs,
      out_specs=pl.BlockSpec((br, n_out), lambda i: (i, 0)),
      out_shape=jax.ShapeDtypeStruct((_N_PAD, n_out), jnp.float32),
  )(*args)


def kernel(x, edge, W1, b1, W2, b2):
  nidx = edge[0].reshape(_NW, _CPW, _CHUNK)
  hidx = edge[1].reshape(_NW, _CPW, _CHUNK)

  xl1 = _tc_matmul(x, W1)
  acc1, d_node, d_hedge = _sc_prop_deg(xl1, nidx, hidx)
  ef1 = _tc_edge_scale(acc1, d_hedge, 128)
  acc2 = _sc_prop128(ef1, hidx, nidx)
  xl2 = _tc_node_out(acc2, d_node, b1, W2, 128)

  acc3 = _sc_prop32(xl2, nidx, hidx)
  ef2 = _tc_edge_scale(acc3, d_hedge, 32)
  acc4 = _sc_prop32(ef2, hidx, nidx)
  out = _tc_node_out(acc4, d_node, b2, None, 32)
  return out[:_N_NODES]


# speedup vs baseline: 24.4856x; 1.1298x over previous
import jax
import jax.numpy as jnp
from jax import lax
from jax.experimental import pallas as pl
from jax.experimental.pallas import tpu as pltpu
from jax.experimental.pallas import tpu_sc as plsc

_N_NODES = 10000
_E_TOTAL = 320000
_N_PAD = 10240
_NC, _NS = 2, 16
_NW = _NC * _NS
_EPW = _E_TOTAL // _NW
_CHUNK = 80
_CPW = _EPW // _CHUNK
_ROWS_PT = _N_PAD // _NS


def _sc_propagate(feat_dim, with_degrees):
  nbuf = 2 if feat_dim > 32 else 5
  ngrp = -(-_CPW // nbuf)
  mesh = plsc.VectorSubcoreMesh(
      core_axis_name="c", subcore_axis_name="s",
      num_cores=_NC, num_subcores=_NS)
  out_type = [jax.ShapeDtypeStruct((_NC * _N_PAD, feat_dim), jnp.float32)]
  if with_degrees:
    out_type += [jax.ShapeDtypeStruct((_NC * _N_PAD,), jnp.float32)] * 2
  scratch = [
      pltpu.VMEM((_CPW, _CHUNK), jnp.int32),
      pltpu.VMEM((_CPW, _CHUNK), jnp.int32),
      pltpu.VMEM((nbuf, _CHUNK, feat_dim), jnp.float32),
      pltpu.VMEM_SHARED((_N_PAD, feat_dim), jnp.float32),
  ] + [pltpu.SemaphoreType.DMA] * nbuf
  if with_degrees:
    scratch += [
        pltpu.VMEM((_CHUNK,), jnp.float32),
        pltpu.VMEM((_ROWS_PT,), jnp.float32),
        pltpu.VMEM_SHARED((_N_PAD,), jnp.float32),
        pltpu.VMEM_SHARED((_N_PAD,), jnp.float32),
    ]

  def body(table, gsrc, ssrc, *rest):
    if with_degrees:
      (acc_out, dg_out, ds_out, gidx2, sidx2, rows3, acc,
       *sems_rest) = rest
      sems = sems_rest[:nbuf]
      ones, zvec, dg, ds = sems_rest[nbuf:]
    else:
      acc_out, gidx2, sidx2, rows3, acc, *sems = rest
    cid = lax.axis_index("c")
    sid = lax.axis_index("s")
    wid = sid * _NC + cid
    zero16 = jnp.zeros((16,), jnp.float32)
    row0 = sid * _ROWS_PT

    pltpu.sync_copy(gsrc.at[wid], gidx2)
    pltpu.sync_copy(ssrc.at[wid], sidx2)

    def zrow(i, carry):
      for j in range(feat_dim // 16):
        rows3[0, i, pl.ds(j * 16, 16)] = zero16
      return carry
    lax.fori_loop(0, _CHUNK, zrow, 0)
    for k in range(_ROWS_PT // _CHUNK):
      pltpu.sync_copy(rows3.at[0], acc.at[pl.ds(row0 + k * _CHUNK, _CHUNK)])
    if with_degrees:
      def zv(i, carry):
        zvec[pl.ds(i * 16, 16)] = zero16
        return carry
      lax.fori_loop(0, _ROWS_PT // 16, zv, 0)
      for j in range(_CHUNK // 16 + 1):
        o = min(j * 16, _CHUNK - 16)
        ones[pl.ds(o, 16)] = jnp.ones((16,), jnp.float32)
      pltpu.sync_copy(zvec, dg.at[pl.ds(row0, _ROWS_PT)])
      pltpu.sync_copy(zvec, ds.at[pl.ds(row0, _ROWS_PT)])
    plsc.subcore_barrier()

    for b in range(nbuf):
      pltpu.async_copy(table.at[gidx2.at[b]], rows3.at[b], sems[b])

    guard = (_CPW % nbuf) != 0

    def grp(g, carry):
      for b in range(nbuf):
        k = g * nbuf + b

        def step():
          pltpu.make_async_copy(
              table.at[gidx2.at[b]], rows3.at[b], sems[b]).wait()
          pltpu.sync_copy(rows3.at[b], acc.at[sidx2.at[k]], add=True)
          if with_degrees:
            pltpu.sync_copy(ones, dg.at[gidx2.at[k]], add=True)
            pltpu.sync_copy(ones, ds.at[sidx2.at[k]], add=True)
          @pl.when(k + nbuf < _CPW)
          def _():
            pltpu.async_copy(
                table.at[gidx2.at[k + nbuf]], rows3.at[b], sems[b])

        if guard:
          pl.when(k < _CPW)(step)
        else:
          step()
      return carry
    lax.fori_loop(0, ngrp, grp, 0)
    plsc.subcore_barrier()

    obase = cid * _N_PAD + row0
    pltpu.sync_copy(acc.at[pl.ds(row0, _ROWS_PT)],
                    acc_out.at[pl.ds(obase, _ROWS_PT)])
    if with_degrees:
      pltpu.sync_copy(dg.at[pl.ds(row0, _ROWS_PT)],
                      dg_out.at[pl.ds(obase, _ROWS_PT)])
      pltpu.sync_copy(ds.at[pl.ds(row0, _ROWS_PT)],
                      ds_out.at[pl.ds(obase, _ROWS_PT)])

  return pl.kernel(
      body,
      out_type=tuple(out_type) if with_degrees else out_type[0],
      mesh=mesh,
      scratch_types=tuple(scratch),
      compiler_params=pltpu.CompilerParams(use_tc_tiling_on_sc=False))


_sc_prop_deg = _sc_propagate(128, True)
_sc_prop128 = _sc_propagate(128, False)
_sc_prop32 = _sc_propagate(32, False)


def _tc_matmul(x, w):
  m, k = x.shape
  n = w.shape[1]
  br = 1000 if m % 1000 == 0 else 1024

  def body(xr, wr, outr):
    outr[...] = jnp.dot(xr[...], wr[...], preferred_element_type=jnp.float32)

  return pl.pallas_call(
      body,
      grid=(m // br,),
      in_specs=[pl.BlockSpec((br, k), lambda i: (i, 0)),
                pl.BlockSpec((k, n), lambda i: (0, 0))],
      out_specs=pl.BlockSpec((br, n), lambda i: (i, 0)),
      out_shape=jax.ShapeDtypeStruct((m, n), jnp.float32),
  )(x, w)


def _tc_edge_scale(acc, deg, feat_dim):
  br = 1024
  acc3 = acc.reshape(_NC, _N_PAD, feat_dim)
  deg3 = deg.reshape(_NC, _N_PAD, 1)

  def body(a0, a1, d0, d1, o):
    s = a0[0] + a1[0]
    b = d0[0] + d1[0]
    o[...] = jnp.where(b > 0, 1.0 / b, 0.0) * s

  return pl.pallas_call(
      body,
      grid=(_N_PAD // br,),
      in_specs=[pl.BlockSpec((1, br, feat_dim), lambda i: (0, i, 0)),
                pl.BlockSpec((1, br, feat_dim), lambda i: (1, i, 0)),
                pl.BlockSpec((1, br, 1), lambda i: (0, i, 0)),
                pl.BlockSpec((1, br, 1), lambda i: (1, i, 0))],
      out_specs=pl.BlockSpec((br, feat_dim), lambda i: (i, 0)),
      out_shape=jax.ShapeDtypeStruct((_N_PAD, feat_dim), jnp.float32),
  )(acc3, acc3, deg3, deg3)


def _tc_node_out(acc, deg, bias, w, feat_dim):
  br = 1024
  acc3 = acc.reshape(_NC, _N_PAD, feat_dim)
  deg3 = deg.reshape(_NC, _N_PAD, 1)
  bias2 = bias.reshape(1, feat_dim)
  n_out = feat_dim if w is None else w.shape[1]

  def body(a0, a1, d0, d1, br_, *rest):
    if w is None:
      (o,) = rest
    else:
      wr, o = rest
    s = a0[0] + a1[0]
    d = d0[0] + d1[0]
    h = jnp.maximum(jnp.where(d > 0, 1.0 / d, 0.0) * s + br_[...], 0.0)
    if w is None:
      o[...] = h
    else:
      o[...] = jnp.dot(h, wr[...], preferred_element_type=jnp.float32)

  in_specs = [pl.BlockSpec((1, br, feat_dim), lambda i: (0, i, 0)),
              pl.BlockSpec((1, br, feat_dim), lambda i: (1, i, 0)),
              pl.BlockSpec((1, br, 1), lambda i: (0, i, 0)),
              pl.BlockSpec((1, br, 1), lambda i: (1, i, 0)),
              pl.BlockSpec((1, feat_dim), lambda i: (0, 0))]
  args = [acc3, acc3, deg3, deg3, bias2]
  if w is not None:
    in_specs.append(pl.BlockSpec(w.shape, lambda i: (0, 0)))
    args.append(w)
  return pl.pallas_call(
      body,
      grid=(_N_PAD // br,),
      in_specs=in_specs,
      out_specs=pl.BlockSpec((br, n_out), lambda i: (i, 0)),
      out_shape=jax.ShapeDtypeStruct((_N_PAD, n_out), jnp.float32),
  )(*args)


def kernel(x, edge, W1, b1, W2, b2):
  nidx = edge[0].reshape(_NW, _CPW, _CHUNK)
  hidx = edge[1].reshape(_NW, _CPW, _CHUNK)

  xl1 = _tc_matmul(x, W1)
  acc1, d_node, d_hedge = _sc_prop_deg(xl1, nidx, hidx)
  ef1 = _tc_edge_scale(acc1, d_hedge, 128)
  acc2 = _sc_prop128(ef1, hidx, nidx)
  xl2 = _tc_node_out(acc2, d_node, b1, W2, 128)

  acc3 = _sc_prop32(xl2, nidx, hidx)
  ef2 = _tc_edge_scale(acc3, d_hedge, 32)
  acc4 = _sc_prop32(ef2, hidx, nidx)
  out = _tc_node_out(acc4, d_node, b2, None, 32)
  return out[:_N_NODES]


# speedup vs baseline: 24.9999x; 1.0210x over previous
import jax
import jax.numpy as jnp
from jax import lax
from jax.experimental import pallas as pl
from jax.experimental.pallas import tpu as pltpu
from jax.experimental.pallas import tpu_sc as plsc

_N_NODES = 10000
_E_TOTAL = 320000
_N_PAD = 10240
_NC, _NS = 2, 16
_NW = _NC * _NS
_EPW = _E_TOTAL // _NW
_CHUNK = 80
_CPW = _EPW // _CHUNK
_ROWS_PT = _N_PAD // _NS


def _sc_propagate(feat_dim, with_degrees):
  nbuf = 2 if feat_dim > 32 else 5
  ngrp = -(-_CPW // nbuf)
  mesh = plsc.VectorSubcoreMesh(
      core_axis_name="c", subcore_axis_name="s",
      num_cores=_NC, num_subcores=_NS)
  out_type = [jax.ShapeDtypeStruct((_NC * _N_PAD, feat_dim), jnp.float32)]
  if with_degrees:
    out_type += [jax.ShapeDtypeStruct((_NC * _N_PAD,), jnp.float32)] * 2
  scratch = [
      pltpu.VMEM((_CPW, _CHUNK), jnp.int32),
      pltpu.VMEM((_CPW, _CHUNK), jnp.int32),
      pltpu.VMEM((nbuf, _CHUNK, feat_dim), jnp.float32),
      pltpu.VMEM_SHARED((_N_PAD, feat_dim), jnp.float32),
  ] + [pltpu.SemaphoreType.DMA] * (nbuf + 10)
  if with_degrees:
    scratch += [
        pltpu.VMEM((_CHUNK,), jnp.float32),
        pltpu.VMEM((_ROWS_PT,), jnp.float32),
        pltpu.VMEM_SHARED((_N_PAD,), jnp.float32),
        pltpu.VMEM_SHARED((_N_PAD,), jnp.float32),
    ]

  def body(table, gsrc, ssrc, *rest):
    if with_degrees:
      (acc_out, dg_out, ds_out, gidx2, sidx2, rows3, acc,
       *sems_rest) = rest
      sems = sems_rest[:nbuf]
      psems = sems_rest[nbuf:nbuf + 10]
      ones, zvec, dg, ds = sems_rest[nbuf + 10:]
    else:
      acc_out, gidx2, sidx2, rows3, acc, *sems_rest = rest
      sems = sems_rest[:nbuf]
      psems = sems_rest[nbuf:nbuf + 10]
    cid = lax.axis_index("c")
    sid = lax.axis_index("s")
    wid = sid * _NC + cid
    zero16 = jnp.zeros((16,), jnp.float32)
    row0 = sid * _ROWS_PT

    pltpu.async_copy(gsrc.at[wid], gidx2, psems[0])
    pltpu.async_copy(ssrc.at[wid], sidx2, psems[1])

    def zrow(i, carry):
      for j in range(feat_dim // 16):
        rows3[0, i, pl.ds(j * 16, 16)] = zero16
      return carry
    lax.fori_loop(0, _CHUNK, zrow, 0)
    nclr = _ROWS_PT // _CHUNK
    for k in range(nclr):
      pltpu.async_copy(rows3.at[0],
                       acc.at[pl.ds(row0 + k * _CHUNK, _CHUNK)], psems[2 + k])
    if with_degrees:
      def zv(i, carry):
        zvec[pl.ds(i * 16, 16)] = zero16
        return carry
      lax.fori_loop(0, _ROWS_PT // 16, zv, 0)
      for j in range(_CHUNK // 16 + 1):
        o = min(j * 16, _CHUNK - 16)
        ones[pl.ds(o, 16)] = jnp.ones((16,), jnp.float32)
      pltpu.async_copy(zvec, dg.at[pl.ds(row0, _ROWS_PT)], sems[0])
      pltpu.async_copy(zvec, ds.at[pl.ds(row0, _ROWS_PT)], sems[1])
    pltpu.make_async_copy(gsrc.at[wid], gidx2, psems[0]).wait()
    pltpu.make_async_copy(ssrc.at[wid], sidx2, psems[1]).wait()
    for k in range(nclr):
      pltpu.make_async_copy(
          rows3.at[0], acc.at[pl.ds(row0 + k * _CHUNK, _CHUNK)],
          psems[2 + k]).wait()
    if with_degrees:
      pltpu.make_async_copy(zvec, dg.at[pl.ds(row0, _ROWS_PT)], sems[0]).wait()
      pltpu.make_async_copy(zvec, ds.at[pl.ds(row0, _ROWS_PT)], sems[1]).wait()
    plsc.subcore_barrier()

    for b in range(nbuf):
      pltpu.async_copy(table.at[gidx2.at[b]], rows3.at[b], sems[b])

    guard = (_CPW % nbuf) != 0

    def grp(g, carry):
      for b in range(nbuf):
        k = g * nbuf + b

        def step():
          pltpu.make_async_copy(
              table.at[gidx2.at[b]], rows3.at[b], sems[b]).wait()
          pltpu.sync_copy(rows3.at[b], acc.at[sidx2.at[k]], add=True)
          if with_degrees:
            pltpu.sync_copy(ones, dg.at[gidx2.at[k]], add=True)
            pltpu.sync_copy(ones, ds.at[sidx2.at[k]], add=True)
          @pl.when(k + nbuf < _CPW)
          def _():
            pltpu.async_copy(
                table.at[gidx2.at[k + nbuf]], rows3.at[b], sems[b])

        if guard:
          pl.when(k < _CPW)(step)
        else:
          step()
      return carry
    lax.fori_loop(0, ngrp, grp, 0)
    plsc.subcore_barrier()

    obase = cid * _N_PAD + row0
    pltpu.sync_copy(acc.at[pl.ds(row0, _ROWS_PT)],
                    acc_out.at[pl.ds(obase, _ROWS_PT)])
    if with_degrees:
      pltpu.sync_copy(dg.at[pl.ds(row0, _ROWS_PT)],
                      dg_out.at[pl.ds(obase, _ROWS_PT)])
      pltpu.sync_copy(ds.at[pl.ds(row0, _ROWS_PT)],
                      ds_out.at[pl.ds(obase, _ROWS_PT)])

  return pl.kernel(
      body,
      out_type=tuple(out_type) if with_degrees else out_type[0],
      mesh=mesh,
      scratch_types=tuple(scratch),
      compiler_params=pltpu.CompilerParams(use_tc_tiling_on_sc=False))


_sc_prop_deg = _sc_propagate(128, True)
_sc_prop128 = _sc_propagate(128, False)
_sc_prop32 = _sc_propagate(32, False)


def _tc_matmul(x, w):
  m, k = x.shape
  n = w.shape[1]
  br = 1000 if m % 1000 == 0 else 1024

  def body(xr, wr, outr):
    outr[...] = jnp.dot(xr[...], wr[...], preferred_element_type=jnp.float32)

  return pl.pallas_call(
      body,
      grid=(m // br,),
      in_specs=[pl.BlockSpec((br, k), lambda i: (i, 0)),
                pl.BlockSpec((k, n), lambda i: (0, 0))],
      out_specs=pl.BlockSpec((br, n), lambda i: (i, 0)),
      out_shape=jax.ShapeDtypeStruct((m, n), jnp.float32),
  )(x, w)


def _tc_edge_scale(acc, deg, feat_dim):
  br = 1024
  acc3 = acc.reshape(_NC, _N_PAD, feat_dim)
  deg3 = deg.reshape(_NC, _N_PAD, 1)

  def body(a0, a1, d0, d1, o):
    s = a0[0] + a1[0]
    b = d0[0] + d1[0]
    o[...] = jnp.where(b > 0, 1.0 / b, 0.0) * s

  return pl.pallas_call(
      body,
      grid=(_N_PAD // br,),
      in_specs=[pl.BlockSpec((1, br, feat_dim), lambda i: (0, i, 0)),
                pl.BlockSpec((1, br, feat_dim), lambda i: (1, i, 0)),
                pl.BlockSpec((1, br, 1), lambda i: (0, i, 0)),
                pl.BlockSpec((1, br, 1), lambda i: (1, i, 0))],
      out_specs=pl.BlockSpec((br, feat_dim), lambda i: (i, 0)),
      out_shape=jax.ShapeDtypeStruct((_N_PAD, feat_dim), jnp.float32),
  )(acc3, acc3, deg3, deg3)


def _tc_node_out(acc, deg, bias, w, feat_dim):
  br = 1024
  acc3 = acc.reshape(_NC, _N_PAD, feat_dim)
  deg3 = deg.reshape(_NC, _N_PAD, 1)
  bias2 = bias.reshape(1, feat_dim)
  n_out = feat_dim if w is None else w.shape[1]

  def body(a0, a1, d0, d1, br_, *rest):
    if w is None:
      (o,) = rest
    else:
      wr, o = rest
    s = a0[0] + a1[0]
    d = d0[0] + d1[0]
    h = jnp.maximum(jnp.where(d > 0, 1.0 / d, 0.0) * s + br_[...], 0.0)
    if w is None:
      o[...] = h
    else:
      o[...] = jnp.dot(h, wr[...], preferred_element_type=jnp.float32)

  in_specs = [pl.BlockSpec((1, br, feat_dim), lambda i: (0, i, 0)),
              pl.BlockSpec((1, br, feat_dim), lambda i: (1, i, 0)),
              pl.BlockSpec((1, br, 1), lambda i: (0, i, 0)),
              pl.BlockSpec((1, br, 1), lambda i: (1, i, 0)),
              pl.BlockSpec((1, feat_dim), lambda i: (0, 0))]
  args = [acc3, acc3, deg3, deg3, bias2]
  if w is not None:
    in_specs.append(pl.BlockSpec(w.shape, lambda i: (0, 0)))
    args.append(w)
  return pl.pallas_call(
      body,
      grid=(_N_PAD // br,),
      in_specs=in_specs,
      out_specs=pl.BlockSpec((br, n_out), lambda i: (i, 0)),
      out_shape=jax.ShapeDtypeStruct((_N_PAD, n_out), jnp.float32),
  )(*args)


def kernel(x, edge, W1, b1, W2, b2):
  nidx = edge[0].reshape(_NW, _CPW, _CHUNK)
  hidx = edge[1].reshape(_NW, _CPW, _CHUNK)

  xl1 = _tc_matmul(x, W1)
  acc1, d_node, d_hedge = _sc_prop_deg(xl1, nidx, hidx)
  ef1 = _tc_edge_scale(acc1, d_hedge, 128)
  acc2 = _sc_prop128(ef1, hidx, nidx)
  xl2 = _tc_node_out(acc2, d_node, b1, W2, 128)

  acc3 = _sc_prop32(xl2, nidx, hidx)
  ef2 = _tc_edge_scale(acc3, d_hedge, 32)
  acc4 = _sc_prop32(ef2, hidx, nidx)
  out = _tc_node_out(acc4, d_node, b2, None, 32)
  return out[:_N_NODES]


# speedup vs baseline: 25.3550x; 1.0142x over previous
import jax
import jax.numpy as jnp
from jax import lax
from jax.experimental import pallas as pl
from jax.experimental.pallas import tpu as pltpu
from jax.experimental.pallas import tpu_sc as plsc

_N_NODES = 10000
_E_TOTAL = 320000
_N_PAD = 10240
_NC, _NS = 2, 16
_NW = _NC * _NS
_EPW = _E_TOTAL // _NW
_CHUNK = 80
_CPW = _EPW // _CHUNK
_ROWS_PT = _N_PAD // _NS


def _sc_propagate(feat_dim, with_degrees):
  nbuf = 2 if feat_dim > 32 else 5
  ngrp = -(-_CPW // nbuf)
  mesh = plsc.VectorSubcoreMesh(
      core_axis_name="c", subcore_axis_name="s",
      num_cores=_NC, num_subcores=_NS)
  out_type = [jax.ShapeDtypeStruct((_NC * _N_PAD, feat_dim), jnp.float32)]
  if with_degrees:
    out_type += [jax.ShapeDtypeStruct((_NC * _N_PAD,), jnp.float32)] * 2
  scratch = [
      pltpu.VMEM((_CPW, _CHUNK), jnp.int32),
      pltpu.VMEM((_CPW, _CHUNK), jnp.int32),
      pltpu.VMEM((nbuf, _CHUNK, feat_dim), jnp.float32),
      pltpu.VMEM_SHARED((_N_PAD, feat_dim), jnp.float32),
  ] + [pltpu.SemaphoreType.DMA] * (nbuf + 10)
  if with_degrees:
    scratch += [
        pltpu.VMEM((_CHUNK,), jnp.float32),
        pltpu.VMEM((_ROWS_PT,), jnp.float32),
        pltpu.VMEM_SHARED((_N_PAD,), jnp.float32),
        pltpu.VMEM_SHARED((_N_PAD,), jnp.float32),
    ]

  def body(table, gsrc, ssrc, *rest):
    if with_degrees:
      (acc_out, dg_out, ds_out, gidx2, sidx2, rows3, acc,
       *sems_rest) = rest
      sems = sems_rest[:nbuf]
      psems = sems_rest[nbuf:nbuf + 10]
      ones, zvec, dg, ds = sems_rest[nbuf + 10:]
    else:
      acc_out, gidx2, sidx2, rows3, acc, *sems_rest = rest
      sems = sems_rest[:nbuf]
      psems = sems_rest[nbuf:nbuf + 10]
    cid = lax.axis_index("c")
    sid = lax.axis_index("s")
    wid = sid * _NC + cid
    zero16 = jnp.zeros((16,), jnp.float32)
    row0 = sid * _ROWS_PT

    pltpu.async_copy(gsrc.at[wid], gidx2, psems[0])
    pltpu.async_copy(ssrc.at[wid], sidx2, psems[1])

    def zrow(i, carry):
      for j in range(feat_dim // 16):
        rows3[0, i, pl.ds(j * 16, 16)] = zero16
      return carry
    lax.fori_loop(0, _CHUNK, zrow, 0)
    nclr = _ROWS_PT // _CHUNK
    for k in range(nclr):
      pltpu.async_copy(rows3.at[0],
                       acc.at[pl.ds(row0 + k * _CHUNK, _CHUNK)], psems[2 + k])
    if with_degrees:
      def zv(i, carry):
        zvec[pl.ds(i * 16, 16)] = zero16
        return carry
      lax.fori_loop(0, _ROWS_PT // 16, zv, 0)
      for j in range(_CHUNK // 16 + 1):
        o = min(j * 16, _CHUNK - 16)
        ones[pl.ds(o, 16)] = jnp.ones((16,), jnp.float32)
      pltpu.async_copy(zvec, dg.at[pl.ds(row0, _ROWS_PT)], sems[0])
      pltpu.async_copy(zvec, ds.at[pl.ds(row0, _ROWS_PT)], sems[1])
    pltpu.make_async_copy(gsrc.at[wid], gidx2, psems[0]).wait()
    pltpu.make_async_copy(ssrc.at[wid], sidx2, psems[1]).wait()
    for k in range(nclr):
      pltpu.make_async_copy(
          rows3.at[0], acc.at[pl.ds(row0 + k * _CHUNK, _CHUNK)],
          psems[2 + k]).wait()
    if with_degrees:
      pltpu.make_async_copy(zvec, dg.at[pl.ds(row0, _ROWS_PT)], sems[0]).wait()
      pltpu.make_async_copy(zvec, ds.at[pl.ds(row0, _ROWS_PT)], sems[1]).wait()
    plsc.subcore_barrier()

    if nbuf >= 5:
      ssems = (psems[8], psems[9])
      for b in range(3):
        pltpu.async_copy(table.at[gidx2.at[b]], rows3.at[b], sems[b])

      def grp(g, carry):
        for b in range(10):
          k = g * 10 + b
          @pl.when(k < _CPW)
          def _():
            @pl.when(k >= 2)
            def _():
              pltpu.make_async_copy(
                  rows3.at[b % 5], acc.at[sidx2.at[k]], ssems[b % 2]).wait()
            @pl.when(k + 3 < _CPW)
            def _():
              pltpu.async_copy(table.at[gidx2.at[k + 3]],
                               rows3.at[(b + 3) % 5], sems[(b + 3) % 5])
            pltpu.make_async_copy(
                table.at[gidx2.at[k]], rows3.at[b % 5], sems[b % 5]).wait()
            pltpu.async_copy(rows3.at[b % 5], acc.at[sidx2.at[k]],
                             ssems[b % 2], add=True)
        return carry
      lax.fori_loop(0, -(-_CPW // 10), grp, 0)
      for k in (_CPW - 2, _CPW - 1):
        pltpu.make_async_copy(
            rows3.at[k % 5], acc.at[sidx2.at[k]], ssems[k % 2]).wait()
    else:
      guard = (_CPW % nbuf) != 0
      if with_degrees:
        dsg, dss = psems[2], psems[3]
      for b in range(nbuf):
        pltpu.async_copy(table.at[gidx2.at[b]], rows3.at[b], sems[b])

      def grp(g, carry):
        for b in range(nbuf):
          k = g * nbuf + b

          def step():
            pltpu.make_async_copy(
                table.at[gidx2.at[b]], rows3.at[b], sems[b]).wait()
            pltpu.sync_copy(rows3.at[b], acc.at[sidx2.at[k]], add=True)
            if with_degrees:
              @pl.when(k >= 1)
              def _():
                pltpu.make_async_copy(ones, dg.at[gidx2.at[k]], dsg).wait()
                pltpu.make_async_copy(ones, ds.at[sidx2.at[k]], dss).wait()
              pltpu.async_copy(ones, dg.at[gidx2.at[k]], dsg, add=True)
              pltpu.async_copy(ones, ds.at[sidx2.at[k]], dss, add=True)
            @pl.when(k + nbuf < _CPW)
            def _():
              pltpu.async_copy(
                  table.at[gidx2.at[k + nbuf]], rows3.at[b], sems[b])

          if guard:
            pl.when(k < _CPW)(step)
          else:
            step()
        return carry
      lax.fori_loop(0, ngrp, grp, 0)
      if with_degrees:
        k = _CPW - 1
        pltpu.make_async_copy(ones, dg.at[gidx2.at[k]], dsg).wait()
        pltpu.make_async_copy(ones, ds.at[sidx2.at[k]], dss).wait()
    plsc.subcore_barrier()

    obase = cid * _N_PAD + row0
    pltpu.sync_copy(acc.at[pl.ds(row0, _ROWS_PT)],
                    acc_out.at[pl.ds(obase, _ROWS_PT)])
    if with_degrees:
      pltpu.sync_copy(dg.at[pl.ds(row0, _ROWS_PT)],
                      dg_out.at[pl.ds(obase, _ROWS_PT)])
      pltpu.sync_copy(ds.at[pl.ds(row0, _ROWS_PT)],
                      ds_out.at[pl.ds(obase, _ROWS_PT)])

  return pl.kernel(
      body,
      out_type=tuple(out_type) if with_degrees else out_type[0],
      mesh=mesh,
      scratch_types=tuple(scratch),
      compiler_params=pltpu.CompilerParams(use_tc_tiling_on_sc=False))


_sc_prop_deg = _sc_propagate(128, True)
_sc_prop128 = _sc_propagate(128, False)
_sc_prop32 = _sc_propagate(32, False)


def _tc_matmul(x, w):
  m, k = x.shape
  n = w.shape[1]
  br = 1000 if m % 1000 == 0 else 1024

  def body(xr, wr, outr):
    outr[...] = jnp.dot(xr[...], wr[...], preferred_element_type=jnp.float32)

  return pl.pallas_call(
      body,
      grid=(m // br,),
      in_specs=[pl.BlockSpec((br, k), lambda i: (i, 0)),
                pl.BlockSpec((k, n), lambda i: (0, 0))],
      out_specs=pl.BlockSpec((br, n), lambda i: (i, 0)),
      out_shape=jax.ShapeDtypeStruct((m, n), jnp.float32),
  )(x, w)


def _tc_edge_scale(acc, deg, feat_dim):
  br = 1024
  acc3 = acc.reshape(_NC, _N_PAD, feat_dim)
  deg3 = deg.reshape(_NC, _N_PAD, 1)

  def body(a0, a1, d0, d1, o):
    s = a0[0] + a1[0]
    b = d0[0] + d1[0]
    o[...] = jnp.where(b > 0, 1.0 / b, 0.0) * s

  return pl.pallas_call(
      body,
      grid=(_N_PAD // br,),
      in_specs=[pl.BlockSpec((1, br, feat_dim), lambda i: (0, i, 0)),
                pl.BlockSpec((1, br, feat_dim), lambda i: (1, i, 0)),
                pl.BlockSpec((1, br, 1), lambda i: (0, i, 0)),
                pl.BlockSpec((1, br, 1), lambda i: (1, i, 0))],
      out_specs=pl.BlockSpec((br, feat_dim), lambda i: (i, 0)),
      out_shape=jax.ShapeDtypeStruct((_N_PAD, feat_dim), jnp.float32),
  )(acc3, acc3, deg3, deg3)


def _tc_node_out(acc, deg, bias, w, feat_dim):
  br = 1024
  acc3 = acc.reshape(_NC, _N_PAD, feat_dim)
  deg3 = deg.reshape(_NC, _N_PAD, 1)
  bias2 = bias.reshape(1, feat_dim)
  n_out = feat_dim if w is None else w.shape[1]

  def body(a0, a1, d0, d1, br_, *rest):
    if w is None:
      (o,) = rest
    else:
      wr, o = rest
    s = a0[0] + a1[0]
    d = d0[0] + d1[0]
    h = jnp.maximum(jnp.where(d > 0, 1.0 / d, 0.0) * s + br_[...], 0.0)
    if w is None:
      o[...] = h
    else:
      o[...] = jnp.dot(h, wr[...], preferred_element_type=jnp.float32)

  in_specs = [pl.BlockSpec((1, br, feat_dim), lambda i: (0, i, 0)),
              pl.BlockSpec((1, br, feat_dim), lambda i: (1, i, 0)),
              pl.BlockSpec((1, br, 1), lambda i: (0, i, 0)),
              pl.BlockSpec((1, br, 1), lambda i: (1, i, 0)),
              pl.BlockSpec((1, feat_dim), lambda i: (0, 0))]
  args = [acc3, acc3, deg3, deg3, bias2]
  if w is not None:
    in_specs.append(pl.BlockSpec(w.shape, lambda i: (0, 0)))
    args.append(w)
  return pl.pallas_call(
      body,
      grid=(_N_PAD // br,),
      in_specs=in_specs,
      out_specs=pl.BlockSpec((br, n_out), lambda i: (i, 0)),
      out_shape=jax.ShapeDtypeStruct((_N_PAD, n_out), jnp.float32),
  )(*args)


def kernel(x, edge, W1, b1, W2, b2):
  nidx = edge[0].reshape(_NW, _CPW, _CHUNK)
  hidx = edge[1].reshape(_NW, _CPW, _CHUNK)

  xl1 = _tc_matmul(x, W1)
  acc1, d_node, d_hedge = _sc_prop_deg(xl1, nidx, hidx)
  ef1 = _tc_edge_scale(acc1, d_hedge, 128)
  acc2 = _sc_prop128(ef1, hidx, nidx)
  xl2 = _tc_node_out(acc2, d_node, b1, W2, 128)

  acc3 = _sc_prop32(xl2, nidx, hidx)
  ef2 = _tc_edge_scale(acc3, d_hedge, 32)
  acc4 = _sc_prop32(ef2, hidx, nidx)
  out = _tc_node_out(acc4, d_node, b2, None, 32)
  return out[:_N_NODES]
